# Initial kernel scaffold; baseline (speedup 1.0000x reference)
import jax
import jax.numpy as jnp
from jax.experimental import pallas as pl


def kernel(species, batch, constant_shifts):
    raise NotImplementedError("write your pallas kernel here")


# speedup vs baseline: 252.9832x; 252.9832x over previous
import functools

import jax
import jax.numpy as jnp
from jax import lax
from jax.experimental import pallas as pl
from jax.experimental.pallas import tpu as pltpu
from jax.experimental.pallas import tpu_sc as plsc

N_ATOMS = 6400000
NUM_EMB = 104
NUM_SEG = 4096
TAB_PAD = 128

NUM_CORES = 2
NUM_SUBCORES = 16
NUM_WORKERS = NUM_CORES * NUM_SUBCORES
CHUNK = N_ATOMS // NUM_WORKERS
BLK = 20000
NUM_BLKS = CHUNK // BLK
VECS = BLK // 16
SEG_SLICE = NUM_SEG // NUM_SUBCORES


def _take16(x, idx):
    return x.at[idx].get(mode="promise_in_bounds")


def _sc_body(species_hbm, batch_hbm, table_hbm, out_hbm,
             table_v, sp_v, b_v, acc_v, red_v, res_v, shared):
    cid = lax.axis_index("c")
    sid = lax.axis_index("s")
    wid = cid * NUM_SUBCORES + sid
    base = wid * CHUNK

    pltpu.sync_copy(table_hbm, table_v)

    zeros16 = jnp.zeros((16,), jnp.float32)

    def zero_body(q, _):
        acc_v[pl.ds(q * 16, 16)] = zeros16
        return _

    lax.fori_loop(0, NUM_SEG // 16, zero_body, None)

    iota = lax.iota(jnp.int32, 16)
    idxm1 = jnp.maximum(iota - 1, 0)
    idxp1 = jnp.minimum(iota + 1, 15)
    is_lane0 = iota == 0
    is_lane15 = iota == 15

    def blk_body(blk, _):
        off = base + blk * BLK
        pltpu.sync_copy(species_hbm.at[pl.ds(off, BLK)], sp_v)
        pltpu.sync_copy(batch_hbm.at[pl.ds(off, BLK)], b_v)

        def vec_body(j, _):
            sp = sp_v[pl.ds(j * 16, 16)]
            b = b_v[pl.ds(j * 16, 16)]
            vals = plsc.load_gather(table_v, [sp])
            c = plsc.cumsum(vals)
            m_start = (b != _take16(b, idxm1)) | is_lane0
            k = plsc.cummax(jnp.where(m_start, iota, 0))
            prev_c = _take16(c, jnp.maximum(k - 1, 0))
            prev_c = jnp.where(k == 0, 0.0, prev_c)
            run_sum = c - prev_c
            m_end = (b != _take16(b, idxp1)) | is_lane15
            plsc.addupdate_scatter(acc_v, [b], run_sum, mask=m_end)
            return _

        lax.fori_loop(0, VECS, vec_body, None)
        return _

    lax.fori_loop(0, NUM_BLKS, blk_body, None)

    pltpu.sync_copy(acc_v, shared.at[sid])
    plsc.subcore_barrier()

    col = sid * SEG_SLICE
    for r in range(NUM_SUBCORES):
        pltpu.sync_copy(shared.at[r, pl.ds(col, SEG_SLICE)],
                        red_v.at[pl.ds(r * SEG_SLICE, SEG_SLICE)])

    def red_body(q, _):
        v = jnp.zeros((16,), jnp.float32)
        for r in range(NUM_SUBCORES):
            v = v + red_v[pl.ds(r * SEG_SLICE + q * 16, 16)]
        res_v[pl.ds(q * 16, 16)] = v
        return _

    lax.fori_loop(0, SEG_SLICE // 16, red_body, None)

    pltpu.sync_copy(res_v, out_hbm.at[cid, pl.ds(col, SEG_SLICE)])


@jax.jit
def _sc_segsum(species, batch, table):
    mesh = plsc.VectorSubcoreMesh(core_axis_name="c", subcore_axis_name="s")
    return pl.kernel(
        _sc_body,
        out_type=jax.ShapeDtypeStruct((NUM_CORES, NUM_SEG), jnp.float32),
        mesh=mesh,
        compiler_params=pltpu.CompilerParams(needs_layout_passes=False),
        scratch_types=[
            pltpu.VMEM((TAB_PAD,), jnp.float32),
            pltpu.VMEM((BLK,), jnp.int32),
            pltpu.VMEM((BLK,), jnp.int32),
            pltpu.VMEM((NUM_SEG,), jnp.float32),
            pltpu.VMEM((NUM_SEG,), jnp.float32),
            pltpu.VMEM((SEG_SLICE,), jnp.float32),
            pltpu.VMEM_SHARED((NUM_SUBCORES, NUM_SEG), jnp.float32),
        ],
    )(species, batch, table)


def _add2_body(p_ref, o_ref):
    o_ref[...] = p_ref[0] + p_ref[1]


@jax.jit
def _add_partials(partials):
    p = partials.reshape(NUM_CORES, 32, 128)
    out = pl.pallas_call(
        _add2_body,
        out_shape=jax.ShapeDtypeStruct((32, 128), jnp.float32),
    )(p)
    return out.reshape(NUM_SEG)


def kernel(species, batch, constant_shifts):
    table = jnp.pad(constant_shifts[:, 0], (0, TAB_PAD - NUM_EMB))
    partials = _sc_segsum(species, batch, table)
    return _add_partials(partials)


# speedup vs baseline: 922.6938x; 3.6473x over previous
import jax
import jax.numpy as jnp
from jax import lax
from jax.experimental import pallas as pl
from jax.experimental.pallas import tpu as pltpu
from jax.experimental.pallas import tpu_sc as plsc

N_ATOMS = 6400000
NUM_EMB = 104
NUM_SEG = 4096
TAB_PAD = 128

NUM_CORES = 2
NUM_SUBCORES = 16
NUM_WORKERS = NUM_CORES * NUM_SUBCORES
CHUNK = N_ATOMS // NUM_WORKERS
BLK = 20000
NUM_BLKS = CHUNK // BLK
VECS = BLK // 16
UNROLL = 5
SEG_SLICE = NUM_SEG // NUM_SUBCORES


def _take16(x, idx):
    return x.at[idx].get(mode="promise_in_bounds")


def _sc_body(species_hbm, batch_hbm, table_hbm, out_hbm,
             table_v, sp0_v, b0_v, sp1_v, b1_v,
             acc0, acc1, acc2, acc3, acc4, red_v, res_v, shared,
             sem_s0, sem_b0, sem_s1, sem_b1):
    accs = [acc0, acc1, acc2, acc3, acc4]
    cid = lax.axis_index("c")
    sid = lax.axis_index("s")
    wid = cid * NUM_SUBCORES + sid
    base = wid * CHUNK

    pltpu.sync_copy(table_hbm, table_v)

    zeros16 = jnp.zeros((16,), jnp.float32)

    def zero_body(q, _):
        for u in range(UNROLL):
            accs[u][pl.ds(q * 16, 16)] = zeros16
        return _

    lax.fori_loop(0, NUM_SEG // 16, zero_body, None)

    iota = lax.iota(jnp.int32, 16)
    idxp1 = jnp.minimum(iota + 1, 15)
    is_lane15 = iota == 15

    bufs = [(sp0_v, b0_v), (sp1_v, b1_v)]
    sems = [(sem_s0, sem_b0), (sem_s1, sem_b1)]
    pending = {}

    def issue(blk):
        pb = blk % 2
        off = base + blk * BLK
        c1 = pltpu.async_copy(species_hbm.at[pl.ds(off, BLK)],
                              bufs[pb][0], sems[pb][0])
        c2 = pltpu.async_copy(batch_hbm.at[pl.ds(off, BLK)],
                              bufs[pb][1], sems[pb][1])
        pending[blk] = (c1, c2)

    issue(0)
    for blk in range(NUM_BLKS):
        if blk + 1 < NUM_BLKS:
            issue(blk + 1)
        c1, c2 = pending.pop(blk)
        c1.wait()
        c2.wait()
        sp_v, b_v = bufs[blk % 2]

        @plsc.parallel_loop(0, VECS, step=UNROLL)
        def vec_body(i):
            for u in range(UNROLL):
                off16 = (i + u) * 16
                sp = sp_v[pl.ds(off16, 16)]
                b = b_v[pl.ds(off16, 16)]
                vals = plsc.load_gather(table_v, [sp])
                c = plsc.cumsum(vals)
                bn = _take16(b, idxp1)
                neq = b != bn
                m1 = neq | is_lane15
                plsc.addupdate_scatter(accs[u], [b], c, mask=m1)
                plsc.addupdate_scatter(accs[u], [bn], 0.0 - c, mask=neq)

    def fold_body(q, _):
        v = accs[0][pl.ds(q * 16, 16)]
        for u in range(1, UNROLL):
            v = v + accs[u][pl.ds(q * 16, 16)]
        accs[0][pl.ds(q * 16, 16)] = v
        return _

    lax.fori_loop(0, NUM_SEG // 16, fold_body, None)

    pltpu.sync_copy(acc0, shared.at[sid])
    plsc.subcore_barrier()

    col = sid * SEG_SLICE
    for r in range(NUM_SUBCORES):
        pltpu.sync_copy(shared.at[r, pl.ds(col, SEG_SLICE)],
                        red_v.at[pl.ds(r * SEG_SLICE, SEG_SLICE)])

    def red_body(q, _):
        v = jnp.zeros((16,), jnp.float32)
        for r in range(NUM_SUBCORES):
            v = v + red_v[pl.ds(r * SEG_SLICE + q * 16, 16)]
        res_v[pl.ds(q * 16, 16)] = v
        return _

    lax.fori_loop(0, SEG_SLICE // 16, red_body, None)

    pltpu.sync_copy(res_v, out_hbm.at[cid, pl.ds(col, SEG_SLICE)])


@jax.jit
def _sc_segsum(species, batch, table):
    mesh = plsc.VectorSubcoreMesh(core_axis_name="c", subcore_axis_name="s")
    return pl.kernel(
        _sc_body,
        out_type=jax.ShapeDtypeStruct((NUM_CORES, NUM_SEG), jnp.float32),
        mesh=mesh,
        compiler_params=pltpu.CompilerParams(needs_layout_passes=False),
        scratch_types=[
            pltpu.VMEM((TAB_PAD,), jnp.float32),
            pltpu.VMEM((BLK,), jnp.int32),
            pltpu.VMEM((BLK,), jnp.int32),
            pltpu.VMEM((BLK,), jnp.int32),
            pltpu.VMEM((BLK,), jnp.int32),
            pltpu.VMEM((NUM_SEG,), jnp.float32),
            pltpu.VMEM((NUM_SEG,), jnp.float32),
            pltpu.VMEM((NUM_SEG,), jnp.float32),
            pltpu.VMEM((NUM_SEG,), jnp.float32),
            pltpu.VMEM((NUM_SEG,), jnp.float32),
            pltpu.VMEM((NUM_SEG,), jnp.float32),
            pltpu.VMEM((SEG_SLICE,), jnp.float32),
            pltpu.VMEM_SHARED((NUM_SUBCORES, NUM_SEG), jnp.float32),
            pltpu.SemaphoreType.DMA,
            pltpu.SemaphoreType.DMA,
            pltpu.SemaphoreType.DMA,
            pltpu.SemaphoreType.DMA,
        ],
    )(species, batch, table)


def _add2_body(p_ref, o_ref):
    o_ref[...] = p_ref[0] + p_ref[1]


@jax.jit
def _add_partials(partials):
    p = partials.reshape(NUM_CORES, 32, 128)
    out = pl.pallas_call(
        _add2_body,
        out_shape=jax.ShapeDtypeStruct((32, 128), jnp.float32),
    )(p)
    return out.reshape(NUM_SEG)


def kernel(species, batch, constant_shifts):
    table = jnp.pad(constant_shifts[:, 0], (0, TAB_PAD - NUM_EMB))
    partials = _sc_segsum(species, batch, table)
    return _add_partials(partials)
